# baseline (speedup 1.0000x reference)
import jax
import jax.numpy as jnp
from jax.experimental import pallas as pl
from jax.experimental.pallas import tpu as pltpu

NUM_CLASSES = 81
B = 4
Q = 300
CT = 41
N2 = 2 * Q
THRESHOLD = 0.05
N_MAX = 100
BG_LOGITS = -5.0
EOS_COEF = 0.1
W_CE = 1.0
W_BBOX = 5.0
W_GIOU = 2.0
W_KD = 1.0
INF = float('inf')


def _tr(x):
    return jnp.transpose(x, (1, 0))


def _fiota(shape, dim):
    return jax.lax.broadcasted_iota(jnp.int32, shape, dim).astype(jnp.float32)


def _dot_raw(a, b):
    return jax.lax.dot_general(a, b, (((1,), (0,)), ((), ())),
                               preferred_element_type=jnp.float32)


def _split3(x):
    hi = x.astype(jnp.bfloat16)
    r1 = x - hi.astype(jnp.float32)
    mid = r1.astype(jnp.bfloat16)
    lo = (r1 - mid.astype(jnp.float32)).astype(jnp.bfloat16)
    return hi, mid, lo


def _sel_dot(onehot, data):
    oh = onehot.astype(jnp.bfloat16)
    hi, mid, lo = _split3(data)
    return (_dot_raw(oh, hi) + _dot_raw(oh, mid)) + _dot_raw(oh, lo)


def _dot_sel(data, onehot):
    oh = onehot.astype(jnp.bfloat16)
    hi, mid, lo = _split3(data)
    return (_dot_raw(hi, oh) + _dot_raw(mid, oh)) + _dot_raw(lo, oh)


def _softmax_parts(x):
    m = jnp.max(x, axis=1, keepdims=True)
    e = jnp.exp(x - m)
    s = jnp.sum(e, axis=1, keepdims=True)
    return e, s


def _giou_terms(x0s, y0s, x1s, y1s, x0t, y0t, x1t, y1t):
    a1 = (x1s - x0s) * (y1s - y0s)
    a2 = (x1t - x0t) * (y1t - y0t)
    iw = jnp.clip(jnp.minimum(x1s, x1t) - jnp.maximum(x0s, x0t), 0.0, None)
    ih = jnp.clip(jnp.minimum(y1s, y1t) - jnp.maximum(y0s, y0t), 0.0, None)
    inter = iw * ih
    union = a1 + a2 - inter
    iou = inter / (union + 1e-8)
    ew = jnp.clip(jnp.maximum(x1s, x1t) - jnp.minimum(x0s, x0t), 0.0, None)
    eh = jnp.clip(jnp.maximum(y1s, y1t) - jnp.minimum(y0s, y0t), 0.0, None)
    ae = ew * eh
    return iou - (ae - union) / (ae + 1e-8)


def _body(ls_ref, bs_ref, lt0_ref, bt0_ref, lt1_ref, bt1_ref, out_ref, c_ref):
    costs = []
    labs = []
    tboxs = []
    sts = []
    valids = []
    lps_parts = []
    bss = []
    nleft0 = jnp.float32(0.0)

    iota_qct = _fiota((Q, CT), 1)
    jcol = jax.lax.broadcasted_iota(jnp.int32, (N2, N2), 0)
    irow = jax.lax.broadcasted_iota(jnp.int32, (N2, N2), 1)
    jlt = jcol < irow
    slot = _fiota((N_MAX, N2), 0)
    iota_n1 = _fiota((N_MAX, 1), 0)
    cls_iota = _fiota((NUM_CLASSES, N_MAX), 0)

    for b in range(B):
        ls = ls_ref[b]
        bs = bs_ref[b]
        lt_list = (lt0_ref[b], lt1_ref[b])
        bt_list = (bt0_ref[b], bt1_ref[b])
        bss.append(bs)

        pms = []
        tgts = []
        for t in (0, 1):
            lt = lt_list[t]
            e, s = _softmax_parts(lt)
            prob = e / s
            pm = jnp.max(prob, axis=1, keepdims=True)
            cm = jnp.min(jnp.where(prob == pm, iota_qct, jnp.float32(1e9)),
                         axis=1, keepdims=True)
            tgt = cm if t == 0 else jnp.where(cm == 0.0, 0.0, cm + 40.0)
            pms.append(pm)
            tgts.append(tgt)
        p_col = jnp.concatenate(pms, axis=0)
        tgt_col = jnp.concatenate(tgts, axis=0)
        mask = (tgt_col != 0.0) & (p_col > THRESHOLD)
        n = jnp.sum(mask.astype(jnp.float32))
        nleft0 = nleft0 + jnp.minimum(n, jnp.float32(N_MAX))
        p_m = jnp.where(mask, p_col, -INF)

        p_row = _tr(p_m)
        beats = (p_m > p_row) | ((p_m == p_row) & jlt)
        rank_row = jnp.sum(beats.astype(jnp.float32), axis=0, keepdims=True)
        S = (rank_row == slot).astype(jnp.float32)

        bbox_t = jnp.concatenate([bt_list[0], bt_list[1]], axis=0)
        soft_t = jnp.concatenate([lt_list[0], lt_list[1]], axis=0)
        T = jnp.concatenate([tgt_col, bbox_t, soft_t], axis=1)
        G = _sel_dot(S, T)
        lab = G[:, 0:1]
        tboxes = G[:, 1:5]
        st = G[:, 5:46]
        labs.append(lab)
        tboxs.append(tboxes)
        sts.append(st)

        valid_col = (iota_n1 < n).astype(jnp.float32)
        valids.append(valid_col)

        mls = jnp.max(ls, axis=1, keepdims=True)
        es = jnp.exp(ls - mls)
        ses = jnp.sum(es, axis=1, keepdims=True)
        lps_parts.append((ls, mls, ses))
        prob_s = es / ses
        lab_row = _tr(lab)
        oh_lab = (cls_iota == lab_row).astype(jnp.float32)
        cost_class = -_dot_sel(prob_s, oh_lab)

        tbT = _tr(tboxes)
        cxt, cyt, wt, ht = (tbT[0:1], tbT[1:2], tbT[2:3], tbT[3:4])
        cxs, cys, ws, hs = (bs[:, 0:1], bs[:, 1:2], bs[:, 2:3], bs[:, 3:4])
        cost_bbox = (jnp.abs(cxs - cxt) + jnp.abs(cys - cyt)
                     + jnp.abs(ws - wt) + jnp.abs(hs - ht))

        giou = _giou_terms(cxs - 0.5 * ws, cys - 0.5 * hs,
                           cxs + 0.5 * ws, cys + 0.5 * hs,
                           cxt - 0.5 * wt, cyt - 0.5 * ht,
                           cxt + 0.5 * wt, cyt + 0.5 * ht)

        cost = W_CE * cost_class + W_BBOX * cost_bbox + W_GIOU * (-giou)
        valid_row = _tr(valid_col)
        costs.append(jnp.where(valid_row > 0.0, cost, INF))

    riota1 = _fiota((Q, N_MAX), 0) + 1.0
    for b in range(B):
        c_ref[b] = costs[b]

    def cond(carry):
        return carry[-1] > 0.5

    def _round(C, rm, nleft):
        rmin = jnp.min(C, axis=1, keepdims=True)
        cmin = jnp.min(C, axis=0, keepdims=True)
        D = (C == rmin) & (rmin == cmin)
        Dsel = jnp.where(D, riota1, 0.0)
        t = jnp.sum(Dsel, axis=0, keepdims=True)
        hit = (t > 0.0) & (cmin < INF)
        rowdead = jnp.max(Dsel, axis=1, keepdims=True) > 0.0
        rm = jnp.where(hit, t - 1.0, rm)
        C = jnp.where(rowdead | hit, INF, C)
        nleft = nleft - jnp.sum(jnp.where(hit, 1.0, 0.0))
        return C, rm, nleft

    def body(carry):
        rms = carry[0:B]
        nleft = carry[-1]
        newrm = []
        for b in range(B):
            C = c_ref[b]
            C, rm, nleft = _round(C, rms[b], nleft)
            C, rm, nleft = _round(C, rm, nleft)
            newrm.append(rm)
            c_ref[b] = C
        return tuple(newrm) + (nleft,)

    rm0 = jnp.full((1, N_MAX), jnp.float32(Q))
    final = jax.lax.while_loop(cond, body, (rm0,) * B + (nleft0,))
    rm_rows = final[0:B]

    acc_ce = jnp.float32(0.0)
    acc_bbox = jnp.float32(0.0)
    acc_giou = jnp.float32(0.0)
    acc_kd = jnp.float32(0.0)
    for b in range(B):
        ls, mls, ses = lps_parts[b]
        bs = bss[b]
        rm_row = rm_rows[b]
        lab = labs[b]
        lab_row = _tr(lab)
        tboxes = tboxs[b]
        st = sts[b]
        valid_col = valids[b]

        logp = (ls - mls) - jnp.log(ses)

        hitq2 = _fiota((Q, N_MAX), 0) == rm_row
        tgtc = jnp.sum(jnp.where(hitq2, lab_row, 0.0), axis=1, keepdims=True)
        oh_t = _fiota((Q, NUM_CLASSES), 1) == tgtc
        ce = -jnp.sum(jnp.where(oh_t, logp, 0.0), axis=1, keepdims=True)
        w = jnp.where(tgtc == 0.0, jnp.float32(EOS_COEF), jnp.float32(1.0))
        acc_ce = acc_ce + jnp.sum(ce * w) / jnp.sum(w)

        rm_col = _tr(rm_row)
        hitq = (_fiota((N_MAX, Q), 1) == rm_col).astype(jnp.float32)
        A = jnp.concatenate([bs, logp], axis=1)
        GA = _sel_dot(hitq, A)
        sb = GA[:, 0:4]
        lps = GA[:, 4:85]

        l1 = jnp.sum(jnp.abs(sb - tboxes), axis=1, keepdims=True)
        acc_bbox = acc_bbox + jnp.sum(l1 * valid_col)

        pgiou = _giou_terms(
            sb[:, 0:1] - 0.5 * sb[:, 2:3], sb[:, 1:2] - 0.5 * sb[:, 3:4],
            sb[:, 0:1] + 0.5 * sb[:, 2:3], sb[:, 1:2] + 0.5 * sb[:, 3:4],
            tboxes[:, 0:1] - 0.5 * tboxes[:, 2:3], tboxes[:, 1:2] - 0.5 * tboxes[:, 3:4],
            tboxes[:, 0:1] + 0.5 * tboxes[:, 2:3], tboxes[:, 1:2] + 0.5 * tboxes[:, 3:4])
        acc_giou = acc_giou + jnp.sum((1.0 - pgiou) * valid_col)

        tid1 = (lab == 0.0) | (lab > 40.0)
        bg = jnp.full((N_MAX, 40), jnp.float32(BG_LOGITS))
        v0 = jnp.concatenate([st, bg], axis=1)
        v1 = jnp.concatenate([st[:, 0:1], bg, st[:, 1:41]], axis=1)
        full = jnp.where(tid1, v1, v0)
        mf = jnp.max(full, axis=1, keepdims=True)
        ef = jnp.exp(full - mf)
        sef = jnp.sum(ef, axis=1, keepdims=True)
        pt = ef / sef
        lpt = (full - mf) - jnp.log(sef)
        acc_kd = acc_kd + jnp.sum(
            jnp.sum(pt * (lpt - lps), axis=1, keepdims=True) * valid_col)

    nb = jnp.maximum(nleft0, jnp.float32(1.0))
    out_ref[0] = (W_CE * (acc_ce / B) + W_BBOX * (acc_bbox / nb)
                  + W_GIOU * (acc_giou / nb) + W_KD * (acc_kd / nb))


@jax.jit
def kernel(pred_logits_s, pred_boxes_s, pred_logits_t0, pred_boxes_t0,
           pred_logits_t1, pred_boxes_t1):
    out = pl.pallas_call(
        _body,
        out_specs=pl.BlockSpec(memory_space=pltpu.SMEM),
        out_shape=jax.ShapeDtypeStruct((1,), jnp.float32),
        scratch_shapes=[pltpu.VMEM((B, Q, N_MAX), jnp.float32)],
    )(pred_logits_s, pred_boxes_s, pred_logits_t0, pred_boxes_t0,
      pred_logits_t1, pred_boxes_t1)
    return out[0]


# speedup vs baseline: 1.0899x; 1.0899x over previous
import jax
import jax.numpy as jnp
from jax.experimental import pallas as pl
from jax.experimental.pallas import tpu as pltpu

NUM_CLASSES = 81
B = 4
Q = 300
CT = 41
N2 = 2 * Q
THRESHOLD = 0.05
N_MAX = 100
BG_LOGITS = -5.0
EOS_COEF = 0.1
W_CE = 1.0
W_BBOX = 5.0
W_GIOU = 2.0
W_KD = 1.0
INF = float('inf')


def _tr(x):
    return jnp.transpose(x, (1, 0))


def _fiota(shape, dim):
    return jax.lax.broadcasted_iota(jnp.int32, shape, dim).astype(jnp.float32)


def _dot_raw(a, b):
    return jax.lax.dot_general(a, b, (((1,), (0,)), ((), ())),
                               preferred_element_type=jnp.float32)


def _split3(x):
    hi = x.astype(jnp.bfloat16)
    r1 = x - hi.astype(jnp.float32)
    mid = r1.astype(jnp.bfloat16)
    lo = (r1 - mid.astype(jnp.float32)).astype(jnp.bfloat16)
    return hi, mid, lo


def _sel_dot(onehot, data):
    oh = onehot.astype(jnp.bfloat16)
    hi, mid, lo = _split3(data)
    return (_dot_raw(oh, hi) + _dot_raw(oh, mid)) + _dot_raw(oh, lo)


def _dot_sel(data, onehot):
    oh = onehot.astype(jnp.bfloat16)
    hi, mid, lo = _split3(data)
    return (_dot_raw(hi, oh) + _dot_raw(mid, oh)) + _dot_raw(lo, oh)


def _softmax_parts(x):
    m = jnp.max(x, axis=1, keepdims=True)
    e = jnp.exp(x - m)
    s = jnp.sum(e, axis=1, keepdims=True)
    return e, s


def _giou_terms(x0s, y0s, x1s, y1s, x0t, y0t, x1t, y1t):
    a1 = (x1s - x0s) * (y1s - y0s)
    a2 = (x1t - x0t) * (y1t - y0t)
    iw = jnp.clip(jnp.minimum(x1s, x1t) - jnp.maximum(x0s, x0t), 0.0, None)
    ih = jnp.clip(jnp.minimum(y1s, y1t) - jnp.maximum(y0s, y0t), 0.0, None)
    inter = iw * ih
    union = a1 + a2 - inter
    iou = inter / (union + 1e-8)
    ew = jnp.clip(jnp.maximum(x1s, x1t) - jnp.minimum(x0s, x0t), 0.0, None)
    eh = jnp.clip(jnp.maximum(y1s, y1t) - jnp.minimum(y0s, y0t), 0.0, None)
    ae = ew * eh
    return iou - (ae - union) / (ae + 1e-8)


def _body(ls_ref, bs_ref, lt0_ref, bt0_ref, lt1_ref, bt1_ref, out_ref, c_ref):
    costs = []
    labs = []
    tboxs = []
    sts = []
    valids = []
    lps_parts = []
    bss = []
    nleft0 = jnp.float32(0.0)

    iota_qct2 = _fiota((N2, CT), 1)
    iota_62 = _fiota((N2, 1), 0)
    jcol = jax.lax.broadcasted_iota(jnp.int32, (N2, N2), 0)
    irow = jax.lax.broadcasted_iota(jnp.int32, (N2, N2), 1)
    jlt = jcol < irow
    slot = _fiota((N_MAX, N2), 0)
    iota_n1 = _fiota((N_MAX, 1), 0)
    cls_iota = _fiota((NUM_CLASSES, N_MAX), 0)

    for b in range(B):
        ls = ls_ref[b]
        bs = bs_ref[b]
        bt_list = (bt0_ref[b], bt1_ref[b])
        bss.append(bs)

        soft_t = jnp.concatenate([lt0_ref[b], lt1_ref[b]], axis=0)
        e, s = _softmax_parts(soft_t)
        prob = e / s
        p_col = jnp.max(prob, axis=1, keepdims=True)
        cm = jnp.min(jnp.where(prob == p_col, iota_qct2, jnp.float32(1e9)),
                     axis=1, keepdims=True)
        tgt_col = jnp.where(iota_62 < Q, cm,
                            jnp.where(cm == 0.0, 0.0, cm + 40.0))
        mask = (tgt_col != 0.0) & (p_col > THRESHOLD)
        n = jnp.sum(mask.astype(jnp.float32))
        nleft0 = nleft0 + jnp.minimum(n, jnp.float32(N_MAX))
        p_m = jnp.where(mask, p_col, -INF)

        p_row = _tr(p_m)
        beats = (p_m > p_row) | ((p_m == p_row) & jlt)
        rank_row = jnp.sum(beats.astype(jnp.float32), axis=0, keepdims=True)
        S = (rank_row == slot).astype(jnp.float32)

        bbox_t = jnp.concatenate([bt_list[0], bt_list[1]], axis=0)
        T = jnp.concatenate([tgt_col, bbox_t, soft_t], axis=1)
        G = _sel_dot(S, T)
        lab = G[:, 0:1]
        tboxes = G[:, 1:5]
        st = G[:, 5:46]
        labs.append(lab)
        tboxs.append(tboxes)
        sts.append(st)

        valid_col = (iota_n1 < n).astype(jnp.float32)
        valids.append(valid_col)

        mls = jnp.max(ls, axis=1, keepdims=True)
        es = jnp.exp(ls - mls)
        ses = jnp.sum(es, axis=1, keepdims=True)
        lps_parts.append((ls, mls, ses))
        prob_s = es / ses
        lab_row = _tr(lab)
        oh_lab = (cls_iota == lab_row).astype(jnp.float32)
        cost_class = -_dot_sel(prob_s, oh_lab)

        tbT = _tr(tboxes)
        cxt, cyt, wt, ht = (tbT[0:1], tbT[1:2], tbT[2:3], tbT[3:4])
        cxs, cys, ws, hs = (bs[:, 0:1], bs[:, 1:2], bs[:, 2:3], bs[:, 3:4])
        cost_bbox = (jnp.abs(cxs - cxt) + jnp.abs(cys - cyt)
                     + jnp.abs(ws - wt) + jnp.abs(hs - ht))

        giou = _giou_terms(cxs - 0.5 * ws, cys - 0.5 * hs,
                           cxs + 0.5 * ws, cys + 0.5 * hs,
                           cxt - 0.5 * wt, cyt - 0.5 * ht,
                           cxt + 0.5 * wt, cyt + 0.5 * ht)

        cost = W_CE * cost_class + W_BBOX * cost_bbox + W_GIOU * (-giou)
        valid_row = _tr(valid_col)
        costs.append(jnp.where(valid_row > 0.0, cost, INF))

    riota1 = _fiota((Q, N_MAX), 0) + 1.0
    for b in range(B):
        c_ref[b] = costs[b]

    def cond(carry):
        return carry[-1] > 0.5

    def _round(C, rm, nleft):
        rmin = jnp.min(C, axis=1, keepdims=True)
        cmin = jnp.min(C, axis=0, keepdims=True)
        D = (C == rmin) & (rmin == cmin)
        Dsel = jnp.where(D, riota1, 0.0)
        t = jnp.sum(Dsel, axis=0, keepdims=True)
        hit = (t > 0.0) & (cmin < INF)
        rowdead = jnp.max(Dsel, axis=1, keepdims=True) > 0.0
        rm = jnp.where(hit, t - 1.0, rm)
        C = jnp.where(rowdead | hit, INF, C)
        nleft = nleft - jnp.sum(jnp.where(hit, 1.0, 0.0))
        return C, rm, nleft

    def body(carry):
        rms = carry[0:B]
        nleft = carry[-1]
        newrm = []
        for b in range(B):
            C = c_ref[b]
            C, rm, nleft = _round(C, rms[b], nleft)
            C, rm, nleft = _round(C, rm, nleft)
            newrm.append(rm)
            c_ref[b] = C
        return tuple(newrm) + (nleft,)

    rm0 = jnp.full((1, N_MAX), jnp.float32(Q))
    final = jax.lax.while_loop(cond, body, (rm0,) * B + (nleft0,))
    rm_rows = final[0:B]

    acc_ce = jnp.float32(0.0)
    acc_bbox = jnp.float32(0.0)
    acc_giou = jnp.float32(0.0)
    acc_kd = jnp.float32(0.0)
    for b in range(B):
        ls, mls, ses = lps_parts[b]
        bs = bss[b]
        rm_row = rm_rows[b]
        lab = labs[b]
        lab_row = _tr(lab)
        tboxes = tboxs[b]
        st = sts[b]
        valid_col = valids[b]

        logp = (ls - mls) - jnp.log(ses)

        hitq2 = _fiota((Q, N_MAX), 0) == rm_row
        tgtc = jnp.sum(jnp.where(hitq2, lab_row, 0.0), axis=1, keepdims=True)
        oh_t = _fiota((Q, NUM_CLASSES), 1) == tgtc
        ce = -jnp.sum(jnp.where(oh_t, logp, 0.0), axis=1, keepdims=True)
        w = jnp.where(tgtc == 0.0, jnp.float32(EOS_COEF), jnp.float32(1.0))
        acc_ce = acc_ce + jnp.sum(ce * w) / jnp.sum(w)

        rm_col = _tr(rm_row)
        hitq = (_fiota((N_MAX, Q), 1) == rm_col).astype(jnp.float32)
        A = jnp.concatenate([bs, logp], axis=1)
        GA = _sel_dot(hitq, A)
        sb = GA[:, 0:4]
        lps = GA[:, 4:85]

        l1 = jnp.sum(jnp.abs(sb - tboxes), axis=1, keepdims=True)
        acc_bbox = acc_bbox + jnp.sum(l1 * valid_col)

        pgiou = _giou_terms(
            sb[:, 0:1] - 0.5 * sb[:, 2:3], sb[:, 1:2] - 0.5 * sb[:, 3:4],
            sb[:, 0:1] + 0.5 * sb[:, 2:3], sb[:, 1:2] + 0.5 * sb[:, 3:4],
            tboxes[:, 0:1] - 0.5 * tboxes[:, 2:3], tboxes[:, 1:2] - 0.5 * tboxes[:, 3:4],
            tboxes[:, 0:1] + 0.5 * tboxes[:, 2:3], tboxes[:, 1:2] + 0.5 * tboxes[:, 3:4])
        acc_giou = acc_giou + jnp.sum((1.0 - pgiou) * valid_col)

        tid1 = (lab == 0.0) | (lab > 40.0)
        bg = jnp.full((N_MAX, 40), jnp.float32(BG_LOGITS))
        v0 = jnp.concatenate([st, bg], axis=1)
        v1 = jnp.concatenate([st[:, 0:1], bg, st[:, 1:41]], axis=1)
        full = jnp.where(tid1, v1, v0)
        mf = jnp.max(full, axis=1, keepdims=True)
        ef = jnp.exp(full - mf)
        sef = jnp.sum(ef, axis=1, keepdims=True)
        pt = ef / sef
        lpt = (full - mf) - jnp.log(sef)
        acc_kd = acc_kd + jnp.sum(
            jnp.sum(pt * (lpt - lps), axis=1, keepdims=True) * valid_col)

    nb = jnp.maximum(nleft0, jnp.float32(1.0))
    out_ref[0] = (W_CE * (acc_ce / B) + W_BBOX * (acc_bbox / nb)
                  + W_GIOU * (acc_giou / nb) + W_KD * (acc_kd / nb))


@jax.jit
def kernel(pred_logits_s, pred_boxes_s, pred_logits_t0, pred_boxes_t0,
           pred_logits_t1, pred_boxes_t1):
    out = pl.pallas_call(
        _body,
        out_specs=pl.BlockSpec(memory_space=pltpu.SMEM),
        out_shape=jax.ShapeDtypeStruct((1,), jnp.float32),
        scratch_shapes=[pltpu.VMEM((B, Q, N_MAX), jnp.float32)],
    )(pred_logits_s, pred_boxes_s, pred_logits_t0, pred_boxes_t0,
      pred_logits_t1, pred_boxes_t1)
    return out[0]


# speedup vs baseline: 1.1547x; 1.0595x over previous
import jax
import jax.numpy as jnp
from jax.experimental import pallas as pl
from jax.experimental.pallas import tpu as pltpu

NUM_CLASSES = 81
B = 4
Q = 300
CT = 41
N2 = 2 * Q
THRESHOLD = 0.05
N_MAX = 100
BG_LOGITS = -5.0
EOS_COEF = 0.1
W_CE = 1.0
W_BBOX = 5.0
W_GIOU = 2.0
W_KD = 1.0
INF = float('inf')


def _tr(x):
    return jnp.transpose(x, (1, 0))


def _fiota(shape, dim):
    return jax.lax.broadcasted_iota(jnp.int32, shape, dim).astype(jnp.float32)


def _dot_raw(a, b):
    return jax.lax.dot_general(a, b, (((1,), (0,)), ((), ())),
                               preferred_element_type=jnp.float32)


def _split3(x):
    hi = x.astype(jnp.bfloat16)
    r1 = x - hi.astype(jnp.float32)
    mid = r1.astype(jnp.bfloat16)
    lo = (r1 - mid.astype(jnp.float32)).astype(jnp.bfloat16)
    return hi, mid, lo


def _sel_dot(onehot, data):
    oh = onehot.astype(jnp.bfloat16)
    hi, mid, lo = _split3(data)
    return (_dot_raw(oh, hi) + _dot_raw(oh, mid)) + _dot_raw(oh, lo)


def _dot_sel(data, onehot):
    oh = onehot.astype(jnp.bfloat16)
    hi, mid, lo = _split3(data)
    return (_dot_raw(hi, oh) + _dot_raw(mid, oh)) + _dot_raw(lo, oh)


def _softmax_parts(x):
    m = jnp.max(x, axis=1, keepdims=True)
    e = jnp.exp(x - m)
    s = jnp.sum(e, axis=1, keepdims=True)
    return e, s


def _giou_terms(x0s, y0s, x1s, y1s, x0t, y0t, x1t, y1t):
    a1 = (x1s - x0s) * (y1s - y0s)
    a2 = (x1t - x0t) * (y1t - y0t)
    iw = jnp.clip(jnp.minimum(x1s, x1t) - jnp.maximum(x0s, x0t), 0.0, None)
    ih = jnp.clip(jnp.minimum(y1s, y1t) - jnp.maximum(y0s, y0t), 0.0, None)
    inter = iw * ih
    union = a1 + a2 - inter
    iou = inter / (union + 1e-8)
    ew = jnp.clip(jnp.maximum(x1s, x1t) - jnp.minimum(x0s, x0t), 0.0, None)
    eh = jnp.clip(jnp.maximum(y1s, y1t) - jnp.minimum(y0s, y0t), 0.0, None)
    ae = ew * eh
    return iou - (ae - union) / (ae + 1e-8)


def _body(ls_ref, bs_ref, lt0_ref, bt0_ref, lt1_ref, bt1_ref, out_ref, c_ref):
    costs = []
    labs = []
    tboxs = []
    sts = []
    valids = []
    lps_parts = []
    bss = []
    nleft0 = jnp.float32(0.0)

    iota_qct2 = _fiota((N2, CT), 1)
    iota_62 = _fiota((N2, 1), 0)
    jcol = jax.lax.broadcasted_iota(jnp.int32, (N2, N2), 0)
    irow = jax.lax.broadcasted_iota(jnp.int32, (N2, N2), 1)
    jlt = jcol < irow
    slot = _fiota((N_MAX, N2), 0)
    iota_n1 = _fiota((N_MAX, 1), 0)
    cls_iota = _fiota((NUM_CLASSES, N_MAX), 0)

    for b in range(B):
        ls = ls_ref[b]
        bs = bs_ref[b]
        bt_list = (bt0_ref[b], bt1_ref[b])
        bss.append(bs)

        soft_t = jnp.concatenate([lt0_ref[b], lt1_ref[b]], axis=0)
        e, s = _softmax_parts(soft_t)
        prob = e / s
        p_col = jnp.max(prob, axis=1, keepdims=True)
        cm = jnp.min(jnp.where(prob == p_col, iota_qct2, jnp.float32(1e9)),
                     axis=1, keepdims=True)
        tgt_col = jnp.where(iota_62 < Q, cm,
                            jnp.where(cm == 0.0, 0.0, cm + 40.0))
        mask = (tgt_col != 0.0) & (p_col > THRESHOLD)
        n = jnp.sum(mask.astype(jnp.float32))
        nleft0 = nleft0 + jnp.minimum(n, jnp.float32(N_MAX))
        p_m = jnp.where(mask, p_col, -INF)

        p_row = _tr(p_m)
        beats = (p_m > p_row) | ((p_m == p_row) & jlt)
        rank_row = jnp.sum(beats.astype(jnp.float32), axis=0, keepdims=True)
        S = (rank_row == slot).astype(jnp.float32)

        bbox_t = jnp.concatenate([bt_list[0], bt_list[1]], axis=0)
        T = jnp.concatenate([tgt_col, bbox_t, soft_t], axis=1)
        G = _sel_dot(S, T)
        lab = G[:, 0:1]
        tboxes = G[:, 1:5]
        st = G[:, 5:46]
        labs.append(lab)
        tboxs.append(tboxes)
        sts.append(st)

        valid_col = (iota_n1 < n).astype(jnp.float32)
        valids.append(valid_col)

        mls = jnp.max(ls, axis=1, keepdims=True)
        es = jnp.exp(ls - mls)
        ses = jnp.sum(es, axis=1, keepdims=True)
        lps_parts.append((ls, mls, ses))
        prob_s = es / ses
        lab_row = _tr(lab)
        oh_lab = (cls_iota == lab_row).astype(jnp.float32)
        cost_class = -_dot_sel(prob_s, oh_lab)

        tbT = _tr(tboxes)
        cxt, cyt, wt, ht = (tbT[0:1], tbT[1:2], tbT[2:3], tbT[3:4])
        cxs, cys, ws, hs = (bs[:, 0:1], bs[:, 1:2], bs[:, 2:3], bs[:, 3:4])
        cost_bbox = (jnp.abs(cxs - cxt) + jnp.abs(cys - cyt)
                     + jnp.abs(ws - wt) + jnp.abs(hs - ht))

        giou = _giou_terms(cxs - 0.5 * ws, cys - 0.5 * hs,
                           cxs + 0.5 * ws, cys + 0.5 * hs,
                           cxt - 0.5 * wt, cyt - 0.5 * ht,
                           cxt + 0.5 * wt, cyt + 0.5 * ht)

        cost = W_CE * cost_class + W_BBOX * cost_bbox + W_GIOU * (-giou)
        valid_row = _tr(valid_col)
        costs.append(jnp.where(valid_row > 0.0, cost, INF))

    riota1 = _fiota((Q, N_MAX), 0) + 1.0
    for b in range(B):
        c_ref[b] = costs[b]

    def cond(carry):
        return carry[-1] > 0.5

    def _round(C, rm, nleft):
        rmin = jnp.min(C, axis=1, keepdims=True)
        cmin = jnp.min(C, axis=0, keepdims=True)
        D = (C == rmin) & (rmin == cmin)
        Dsel = jnp.where(D, riota1, 0.0)
        t = jnp.sum(Dsel, axis=0, keepdims=True)
        hit = (t > 0.0) & (cmin < INF)
        rowdead = jnp.max(Dsel, axis=1, keepdims=True) > 0.0
        rm = jnp.where(hit, t - 1.0, rm)
        C = jnp.where(rowdead | hit, INF, C)
        nleft = nleft - jnp.sum(jnp.where(hit, 1.0, 0.0))
        return C, rm, nleft

    def body(carry):
        rms = carry[0:B]
        nleft = carry[-1]
        newrm = []
        for b in range(B):
            C = c_ref[b]
            C, rm, nleft = _round(C, rms[b], nleft)
            C, rm, nleft = _round(C, rm, nleft)
            newrm.append(rm)
            c_ref[b] = C
        return tuple(newrm) + (nleft,)

    rm0 = jnp.full((1, N_MAX), jnp.float32(Q))
    final = jax.lax.while_loop(cond, body, (rm0,) * B + (nleft0,))
    rm_rows = final[0:B]

    acc_ce = jnp.float32(0.0)
    GAs = []
    for b in range(B):
        ls, mls, ses = lps_parts[b]
        bs = bss[b]
        rm_row = rm_rows[b]
        lab_row = _tr(labs[b])

        logp = (ls - mls) - jnp.log(ses)

        hitq2 = _fiota((Q, N_MAX), 0) == rm_row
        tgtc = jnp.sum(jnp.where(hitq2, lab_row, 0.0), axis=1, keepdims=True)
        oh_t = _fiota((Q, NUM_CLASSES), 1) == tgtc
        ce = -jnp.sum(jnp.where(oh_t, logp, 0.0), axis=1, keepdims=True)
        w = jnp.where(tgtc == 0.0, jnp.float32(EOS_COEF), jnp.float32(1.0))
        acc_ce = acc_ce + jnp.sum(ce * w) / jnp.sum(w)

        rm_col = _tr(rm_row)
        hitq = (_fiota((N_MAX, Q), 1) == rm_col).astype(jnp.float32)
        A = jnp.concatenate([bs, logp], axis=1)
        GAs.append(_sel_dot(hitq, A))

    GA = jnp.concatenate(GAs, axis=0)
    sb = GA[:, 0:4]
    lps = GA[:, 4:85]
    tboxes = jnp.concatenate(tboxs, axis=0)
    st = jnp.concatenate(sts, axis=0)
    lab = jnp.concatenate(labs, axis=0)
    valid_col = jnp.concatenate(valids, axis=0)

    l1 = jnp.sum(jnp.abs(sb - tboxes), axis=1, keepdims=True)
    acc_bbox = jnp.sum(l1 * valid_col)

    pgiou = _giou_terms(
        sb[:, 0:1] - 0.5 * sb[:, 2:3], sb[:, 1:2] - 0.5 * sb[:, 3:4],
        sb[:, 0:1] + 0.5 * sb[:, 2:3], sb[:, 1:2] + 0.5 * sb[:, 3:4],
        tboxes[:, 0:1] - 0.5 * tboxes[:, 2:3], tboxes[:, 1:2] - 0.5 * tboxes[:, 3:4],
        tboxes[:, 0:1] + 0.5 * tboxes[:, 2:3], tboxes[:, 1:2] + 0.5 * tboxes[:, 3:4])
    acc_giou = jnp.sum((1.0 - pgiou) * valid_col)

    tid1 = (lab == 0.0) | (lab > 40.0)
    bg = jnp.full((B * N_MAX, 40), jnp.float32(BG_LOGITS))
    v0 = jnp.concatenate([st, bg], axis=1)
    v1 = jnp.concatenate([st[:, 0:1], bg, st[:, 1:41]], axis=1)
    full = jnp.where(tid1, v1, v0)
    mf = jnp.max(full, axis=1, keepdims=True)
    ef = jnp.exp(full - mf)
    sef = jnp.sum(ef, axis=1, keepdims=True)
    pt = ef / sef
    lpt = (full - mf) - jnp.log(sef)
    acc_kd = jnp.sum(jnp.sum(pt * (lpt - lps), axis=1, keepdims=True) * valid_col)

    nb = jnp.maximum(nleft0, jnp.float32(1.0))
    out_ref[0] = (W_CE * (acc_ce / B) + W_BBOX * (acc_bbox / nb)
                  + W_GIOU * (acc_giou / nb) + W_KD * (acc_kd / nb))


@jax.jit
def kernel(pred_logits_s, pred_boxes_s, pred_logits_t0, pred_boxes_t0,
           pred_logits_t1, pred_boxes_t1):
    out = pl.pallas_call(
        _body,
        out_specs=pl.BlockSpec(memory_space=pltpu.SMEM),
        out_shape=jax.ShapeDtypeStruct((1,), jnp.float32),
        scratch_shapes=[pltpu.VMEM((B, Q, N_MAX), jnp.float32)],
    )(pred_logits_s, pred_boxes_s, pred_logits_t0, pred_boxes_t0,
      pred_logits_t1, pred_boxes_t1)
    return out[0]
